# baseline (speedup 1.0000x reference)
import functools

import jax
import jax.numpy as jnp
from jax import lax
from jax.experimental import pallas as pl
from jax.experimental.pallas import tpu as pltpu
from jax.experimental.pallas import tpu_sc as plsc

D = 64
EPS = 1e-5
NC, NS = 2, 16
NW = NC * NS
GCH = 40


def _sc_gather(nodes, idx):
    e = idx.shape[0]
    per_w = e // NW
    n_ch = per_w // GCH
    mesh = plsc.VectorSubcoreMesh(core_axis_name="c", subcore_axis_name="s")

    @functools.partial(
        pl.kernel,
        mesh=mesh,
        out_type=jax.ShapeDtypeStruct((e, 2 * D), jnp.float32),
        scratch_types=[
            pltpu.VMEM((per_w,), jnp.int32),
            pltpu.VMEM((GCH, 2 * D), jnp.float32),
            pltpu.VMEM((GCH, 2 * D), jnp.float32),
            pltpu.SemaphoreType.DMA,
            pltpu.SemaphoreType.DMA,
        ],
    )
    def k(nodes_hbm, idx_hbm, out_hbm, idx_v, rows_a, rows_b, sga, sgb):
        wid = lax.axis_index("s") * NC + lax.axis_index("c")
        base = wid * per_w
        pltpu.sync_copy(idx_hbm.at[pl.ds(base, per_w)], idx_v)
        pltpu.async_copy(nodes_hbm.at[idx_v.at[pl.ds(0, GCH)]], rows_a, sga)

        def drain(rows, sem):
            pltpu.make_async_copy(nodes_hbm.at[pl.ds(0, GCH)], rows, sem).wait()

        @pl.loop(0, n_ch)
        def _(j):
            @pl.when(j % 2 == 0)
            def _():
                @pl.when(j + 1 < n_ch)
                def _():
                    pltpu.async_copy(
                        nodes_hbm.at[idx_v.at[pl.ds((j + 1) * GCH, GCH)]],
                        rows_b, sgb)
                drain(rows_a, sga)
                pltpu.sync_copy(rows_a, out_hbm.at[pl.ds(base + j * GCH, GCH)])

            @pl.when(j % 2 == 1)
            def _():
                @pl.when(j + 1 < n_ch)
                def _():
                    pltpu.async_copy(
                        nodes_hbm.at[idx_v.at[pl.ds((j + 1) * GCH, GCH)]],
                        rows_a, sga)
                drain(rows_b, sgb)
                pltpu.sync_copy(rows_b, out_hbm.at[pl.ds(base + j * GCH, GCH)])

    return k(nodes, idx)


def _sc_scatter_add(msg, idx, zeros, n_pad):
    e = idx.shape[0]
    per_w = e // NW
    n_ch = per_w // GCH
    rps = n_pad // NS
    mesh = plsc.VectorSubcoreMesh(core_axis_name="c", subcore_axis_name="s")

    @functools.partial(
        pl.kernel,
        mesh=mesh,
        out_type=jax.ShapeDtypeStruct((2 * n_pad, 2 * D), jnp.float32),
        scratch_types=[
            pltpu.VMEM((n_ch, GCH), jnp.int32),
            pltpu.VMEM((GCH, 2 * D), jnp.float32),
            pltpu.VMEM((GCH, 2 * D), jnp.float32),
            pltpu.VMEM_SHARED((n_pad, 2 * D), jnp.float32),
            pltpu.SemaphoreType.DMA,
            pltpu.SemaphoreType.DMA,
        ],
    )
    def k(msg_hbm, idx_hbm, zeros_hbm, out_hbm, idx_v, rows_a, rows_b, shared,
          sma, smb):
        cid = lax.axis_index("c")
        sid = lax.axis_index("s")
        wid = sid * NC + cid
        r0 = sid * rps

        pltpu.sync_copy(idx_hbm.at[wid], idx_v)
        pltpu.sync_copy(zeros_hbm.at[pl.ds(r0, rps)], shared.at[pl.ds(r0, rps)])
        plsc.subcore_barrier()

        base = wid * per_w
        pltpu.async_copy(msg_hbm.at[pl.ds(base, GCH)], rows_a, sma)

        def drain(rows, sem):
            pltpu.make_async_copy(msg_hbm.at[pl.ds(0, GCH)], rows, sem).wait()

        @pl.loop(0, n_ch)
        def _(j):
            @pl.when(j % 2 == 0)
            def _():
                @pl.when(j + 1 < n_ch)
                def _():
                    pltpu.async_copy(
                        msg_hbm.at[pl.ds(base + (j + 1) * GCH, GCH)],
                        rows_b, smb)
                drain(rows_a, sma)
                pltpu.sync_copy(rows_a, shared.at[idx_v.at[j]], add=True)

            @pl.when(j % 2 == 1)
            def _():
                @pl.when(j + 1 < n_ch)
                def _():
                    pltpu.async_copy(
                        msg_hbm.at[pl.ds(base + (j + 1) * GCH, GCH)],
                        rows_a, sma)
                drain(rows_b, smb)
                pltpu.sync_copy(rows_b, shared.at[idx_v.at[j]], add=True)

        plsc.subcore_barrier()
        pltpu.sync_copy(shared.at[pl.ds(r0, rps)],
                        out_hbm.at[pl.ds(cid * n_pad + r0, rps)])

    return k(msg, idx.reshape(NW, n_ch, GCH), zeros)


def _matrelu_body(x_ref, w_ref, b_ref, o_ref):
    o_ref[...] = jnp.maximum(
        jnp.dot(x_ref[...], w_ref[...], preferred_element_type=jnp.float32)
        + b_ref[...], 0.0)


def _matrelu(x, w, b, out_dim, rb=8000):
    n, k = x.shape
    return pl.pallas_call(
        _matrelu_body,
        grid=(n // rb,),
        in_specs=[
            pl.BlockSpec((rb, k), lambda i: (i, 0)),
            pl.BlockSpec((k, out_dim), lambda i: (0, 0)),
            pl.BlockSpec((1, out_dim), lambda i: (0, 0)),
        ],
        out_specs=pl.BlockSpec((rb, out_dim), lambda i: (i, 0)),
        out_shape=jax.ShapeDtypeStruct((n, out_dim), jnp.float32),
    )(x, w, b.reshape(1, out_dim))


def _lin0_body(x_ref, w_ref, b_ref, o_ref, osc_ref):
    res = jnp.maximum(
        jnp.dot(x_ref[...], w_ref[...], preferred_element_type=jnp.float32)
        + b_ref[...], 0.0)
    o_ref[...] = res
    osc_ref[...] = jnp.concatenate([res, res], axis=1)


def _lin0(x, w, b):
    n = x.shape[0]
    return pl.pallas_call(
        _lin0_body,
        out_shape=(jax.ShapeDtypeStruct((n, D), jnp.float32),
                   jax.ShapeDtypeStruct((n, 2 * D), jnp.float32)),
    )(x, w, b.reshape(1, D))


def _msg_body(z_ref, hs_ref, w2r_ref, bm_ref, msg_ref):
    z = z_ref[...].astype(jnp.bfloat16)
    hs16 = hs_ref[:, :D].astype(jnp.bfloat16)
    u = jnp.concatenate([z[:, c:c + 1] * hs16 for c in range(32)], axis=1)
    acc = (jnp.dot(u, w2r_ref[...], preferred_element_type=jnp.float32)
           + jnp.dot(hs16, bm_ref[...], preferred_element_type=jnp.float32))
    msg_ref[...] = jnp.concatenate([acc, jnp.zeros_like(acc)], axis=1)


def _msg(z, hs, w2r, bmat, eb=1600):
    e = z.shape[0]
    return pl.pallas_call(
        _msg_body,
        grid=(e // eb,),
        in_specs=[
            pl.BlockSpec((eb, 32), lambda i: (i, 0)),
            pl.BlockSpec((eb, 2 * D), lambda i: (i, 0)),
            pl.BlockSpec((2048, D), lambda i: (0, 0)),
            pl.BlockSpec((D, D), lambda i: (0, 0)),
        ],
        out_specs=pl.BlockSpec((eb, 2 * D), lambda i: (i, 0)),
        out_shape=jax.ShapeDtypeStruct((e, 2 * D), jnp.float32),
    )(z, hs, w2r.astype(jnp.bfloat16), bmat.astype(jnp.bfloat16))


def _upd_body(agg2_ref, out_ref, wt_ref, wb_ref, cb_ref, mb_ref, new_ref,
              nsc_ref):
    n = out_ref.shape[0]
    n_pad = agg2_ref.shape[0] // 2
    agg = agg2_ref[:n, :D] + agg2_ref[n_pad:n_pad + n, :D]
    out = out_ref[...]
    m = jnp.maximum(agg + out + cb_ref[...], 0.0)
    res = (jnp.dot(m, wt_ref[...], preferred_element_type=jnp.float32)
           + jnp.dot(out, wb_ref[...], preferred_element_type=jnp.float32)
           + mb_ref[...])
    new_ref[...] = res
    nsc_ref[...] = jnp.concatenate([res, res], axis=1)


def _upd(agg2, out, wt, wb, cb, mb):
    n = out.shape[0]
    return pl.pallas_call(
        _upd_body,
        out_shape=(jax.ShapeDtypeStruct((n, D), jnp.float32),
                   jax.ShapeDtypeStruct((n, 2 * D), jnp.float32)),
    )(agg2, out, wt, wb, cb.reshape(1, D), mb.reshape(1, D))


def _ln_rows(x, g, b):
    mu = jnp.mean(x, axis=1, keepdims=True)
    xc = x - mu
    var = jnp.mean(xc * xc, axis=1, keepdims=True)
    return xc * lax.rsqrt(var + EPS) * g + b


def _proj_body(out_ref, init_ref, g_ref, b_ref, wq_ref, wk_ref, wv_ref,
               q_ref, k_ref, v_ref):
    h = _ln_rows(out_ref[...] + init_ref[...], g_ref[...], b_ref[...])
    q_ref[...] = jnp.maximum(
        jnp.dot(h, wq_ref[...], preferred_element_type=jnp.float32), 0.0)
    k_ref[...] = jnp.maximum(
        jnp.dot(h, wk_ref[...], preferred_element_type=jnp.float32), 0.0)
    v_ref[...] = jnp.dot(h, wv_ref[...], preferred_element_type=jnp.float32)


def _proj(out, init, g, b, wq, wk, wv):
    n = out.shape[0]
    sh = jax.ShapeDtypeStruct((n, D), jnp.float32)
    return pl.pallas_call(
        _proj_body,
        out_shape=(sh, sh, sh),
    )(out, init, g.reshape(1, D), b.reshape(1, D), wq, wk, wv)


def _attn_row_body(q_ref, k_ref, v_ref, mask_ref, g_ref, b_ref, c_ref):
    q = q_ref[...]
    s = lax.dot_general(q, k_ref[...], (((1,), (1,)), ((), ())),
                        preferred_element_type=jnp.float32)
    mask = mask_ref[...]
    a = mask * s - 1000.0 * (1.0 - mask)
    amax = jnp.max(a, axis=1, keepdims=True)
    ex = jnp.exp(a - amax)
    p = ex / jnp.sum(ex, axis=1, keepdims=True)
    c = jnp.dot(p, v_ref[...], preferred_element_type=jnp.float32)
    c_ref[...] = _ln_rows(c, g_ref[...], b_ref[...])


def _attn_row(q, k, v, mask, g, b, bq=200):
    nq = q.shape[0]
    nk = k.shape[0]
    return pl.pallas_call(
        _attn_row_body,
        grid=(nq // bq,),
        in_specs=[
            pl.BlockSpec((bq, D), lambda i: (i, 0)),
            pl.BlockSpec((nk, D), lambda i: (0, 0)),
            pl.BlockSpec((nk, D), lambda i: (0, 0)),
            pl.BlockSpec((bq, nk), lambda i: (i, 0)),
            pl.BlockSpec((1, D), lambda i: (0, 0)),
            pl.BlockSpec((1, D), lambda i: (0, 0)),
        ],
        out_specs=pl.BlockSpec((bq, D), lambda i: (i, 0)),
        out_shape=jax.ShapeDtypeStruct((nq, D), jnp.float32),
    )(q, k, v, mask, g.reshape(1, D), b.reshape(1, D))


def _attn_col_body(q_ref, k_ref, v_ref, mask_ref, g_ref, b_ref, c_ref):
    s = lax.dot_general(k_ref[...], q_ref[...], (((1,), (1,)), ((), ())),
                        preferred_element_type=jnp.float32)
    mask = mask_ref[...]
    a = mask * s - 1000.0 * (1.0 - mask)
    amax = jnp.max(a, axis=0, keepdims=True)
    ex = jnp.exp(a - amax)
    p = ex / jnp.sum(ex, axis=0, keepdims=True)
    c = lax.dot_general(p, v_ref[...], (((0,), (0,)), ((), ())),
                        preferred_element_type=jnp.float32)
    c_ref[...] = _ln_rows(c, g_ref[...], b_ref[...])


def _attn_col(q, k, v, mask, g, b, bq=256):
    nq = q.shape[0]
    nk = k.shape[0]
    return pl.pallas_call(
        _attn_col_body,
        grid=(nq // bq,),
        in_specs=[
            pl.BlockSpec((bq, D), lambda i: (i, 0)),
            pl.BlockSpec((nk, D), lambda i: (0, 0)),
            pl.BlockSpec((nk, D), lambda i: (0, 0)),
            pl.BlockSpec((nk, bq), lambda i: (0, i)),
            pl.BlockSpec((1, D), lambda i: (0, 0)),
            pl.BlockSpec((1, D), lambda i: (0, 0)),
        ],
        out_specs=pl.BlockSpec((bq, D), lambda i: (i, 0)),
        out_shape=jax.ShapeDtypeStruct((nq, D), jnp.float32),
    )(q, k, v, mask, g.reshape(1, D), b.reshape(1, D))


def _two_branches(x_l, ef_l, ei_l, x_r, ef_r, ei_r, p, w2r, bmat, wt, wb):
    z_l = _matrelu(ef_l, p['en_W1'], p['en_b1'], 32)
    z_r = _matrelu(ef_r, p['en_W1'], p['en_b1'], 32)
    out_l, sc_l = _lin0(x_l, p['lin0_W'], p['lin0_b'])
    out_r, sc_r = _lin0(x_r, p['lin0_W'], p['lin0_b'])
    n_l, n_r = x_l.shape[0], x_r.shape[0]
    np_l, np_r = -(-n_l // 128) * 128, -(-n_r // 128) * 128
    zeros_l = jnp.zeros((np_l, 2 * D), jnp.float32)
    zeros_r = jnp.zeros((np_r, 2 * D), jnp.float32)
    for _ in range(3):
        hs_l = _sc_gather(sc_l, ei_l[0])
        hs_r = _sc_gather(sc_r, ei_r[0])
        msg_l = _msg(z_l, hs_l, w2r, bmat)
        msg_r = _msg(z_r, hs_r, w2r, bmat)
        agg_l = _sc_scatter_add(msg_l, ei_l[1], zeros_l, np_l)
        agg_r = _sc_scatter_add(msg_r, ei_r[1], zeros_r, np_r)
        out_l, sc_l = _upd(agg_l, out_l, wt, wb, p['conv_b'], p['msg_b'])
        out_r, sc_r = _upd(agg_r, out_r, wt, wb, p['conv_b'], p['msg_b'])
    return out_l, out_r


def kernel(lig_n_feat, lig_e_feat, lig_edge_index, rec_n_feat, rec_e_feat,
           rec_edge_index, mask, params):
    p = params
    w2r = p['en_W2'].reshape(2048, D)
    bmat = p['en_b2'].reshape(D, D)
    wt, wb = p['msg_W'][:D], p['msg_W'][D:]

    out_l, out_r = _two_branches(lig_n_feat, lig_e_feat, lig_edge_index,
                                 rec_n_feat, rec_e_feat, rec_edge_index,
                                 p, w2r, bmat, wt, wb)

    q_l, k_l, v_l = _proj(out_l, lig_n_feat, p['ln_lig_g'], p['ln_lig_b'],
                          p['Wq_lig'], p['Wk_lig'], p['Wv_lig'])
    q_r, k_r, v_r = _proj(out_r, rec_n_feat, p['ln_rec_g'], p['ln_rec_b'],
                          p['Wq_rec'], p['Wk_rec'], p['Wv_rec'])

    n_rec = rec_n_feat.shape[0]
    nr_pad = -(-n_rec // 128) * 128
    padr = nr_pad - n_rec
    mask_p = jnp.pad(mask, ((0, 0), (0, padr)))
    c_l = _attn_row(q_l,
                    jnp.pad(k_r, ((0, padr), (0, 0))),
                    jnp.pad(v_r, ((0, padr), (0, 0))),
                    mask_p, p['ln_lig_g'], p['ln_lig_b'])
    c_r = _attn_col(jnp.pad(q_r, ((0, padr), (0, 0))),
                    k_l, v_l, mask_p,
                    p['ln_rec_g'], p['ln_rec_b'])[:n_rec]
    return jnp.concatenate([c_l, c_r], axis=0)


# speedup vs baseline: 1.0450x; 1.0450x over previous
import functools

import jax
import jax.numpy as jnp
from jax import lax
from jax.experimental import pallas as pl
from jax.experimental.pallas import tpu as pltpu
from jax.experimental.pallas import tpu_sc as plsc

D = 64
EPS = 1e-5
NC, NS = 2, 16
NW = NC * NS
GCH = 40


def _sc_gather(nodes, idx):
    e = idx.shape[0]
    per_w = e // NW
    n_ch = per_w // GCH
    mesh = plsc.VectorSubcoreMesh(core_axis_name="c", subcore_axis_name="s")

    @functools.partial(
        pl.kernel,
        mesh=mesh,
        out_type=jax.ShapeDtypeStruct((e, 2 * D), jnp.float32),
        scratch_types=[
            pltpu.VMEM((per_w,), jnp.int32),
            pltpu.VMEM((GCH, 2 * D), jnp.float32),
            pltpu.VMEM((GCH, 2 * D), jnp.float32),
            pltpu.SemaphoreType.DMA,
            pltpu.SemaphoreType.DMA,
        ],
    )
    def k(nodes_hbm, idx_hbm, out_hbm, idx_v, rows_a, rows_b, sga, sgb):
        wid = lax.axis_index("s") * NC + lax.axis_index("c")
        base = wid * per_w
        pltpu.sync_copy(idx_hbm.at[pl.ds(base, per_w)], idx_v)
        pltpu.async_copy(nodes_hbm.at[idx_v.at[pl.ds(0, GCH)]], rows_a, sga)

        def drain(rows, sem):
            pltpu.make_async_copy(nodes_hbm.at[pl.ds(0, GCH)], rows, sem).wait()

        @pl.loop(0, n_ch)
        def _(j):
            @pl.when(j % 2 == 0)
            def _():
                @pl.when(j + 1 < n_ch)
                def _():
                    pltpu.async_copy(
                        nodes_hbm.at[idx_v.at[pl.ds((j + 1) * GCH, GCH)]],
                        rows_b, sgb)
                drain(rows_a, sga)
                pltpu.sync_copy(rows_a, out_hbm.at[pl.ds(base + j * GCH, GCH)])

            @pl.when(j % 2 == 1)
            def _():
                @pl.when(j + 1 < n_ch)
                def _():
                    pltpu.async_copy(
                        nodes_hbm.at[idx_v.at[pl.ds((j + 1) * GCH, GCH)]],
                        rows_a, sga)
                drain(rows_b, sgb)
                pltpu.sync_copy(rows_b, out_hbm.at[pl.ds(base + j * GCH, GCH)])

    return k(nodes, idx)


def _sc_scatter_add(msg, idx, zeros, n_pad):
    e = idx.shape[0]
    per_w = e // NW
    n_ch = per_w // GCH
    rps = n_pad // NS
    mesh = plsc.VectorSubcoreMesh(core_axis_name="c", subcore_axis_name="s")

    @functools.partial(
        pl.kernel,
        mesh=mesh,
        out_type=jax.ShapeDtypeStruct((2 * n_pad, 2 * D), jnp.float32),
        scratch_types=[
            pltpu.VMEM((n_ch, GCH), jnp.int32),
            pltpu.VMEM((GCH, 2 * D), jnp.float32),
            pltpu.VMEM((GCH, 2 * D), jnp.float32),
            pltpu.VMEM_SHARED((n_pad, 2 * D), jnp.float32),
            pltpu.SemaphoreType.DMA,
            pltpu.SemaphoreType.DMA,
        ],
    )
    def k(msg_hbm, idx_hbm, zeros_hbm, out_hbm, idx_v, rows_a, rows_b, shared,
          sma, smb):
        cid = lax.axis_index("c")
        sid = lax.axis_index("s")
        wid = sid * NC + cid
        r0 = sid * rps

        pltpu.sync_copy(idx_hbm.at[wid], idx_v)
        pltpu.sync_copy(zeros_hbm.at[pl.ds(r0, rps)], shared.at[pl.ds(r0, rps)])
        plsc.subcore_barrier()

        base = wid * per_w
        pltpu.async_copy(msg_hbm.at[pl.ds(base, GCH)], rows_a, sma)

        def drain(rows, sem):
            pltpu.make_async_copy(msg_hbm.at[pl.ds(0, GCH)], rows, sem).wait()

        @pl.loop(0, n_ch)
        def _(j):
            @pl.when(j % 2 == 0)
            def _():
                @pl.when(j + 1 < n_ch)
                def _():
                    pltpu.async_copy(
                        msg_hbm.at[pl.ds(base + (j + 1) * GCH, GCH)],
                        rows_b, smb)
                drain(rows_a, sma)
                pltpu.sync_copy(rows_a, shared.at[idx_v.at[j]], add=True)

            @pl.when(j % 2 == 1)
            def _():
                @pl.when(j + 1 < n_ch)
                def _():
                    pltpu.async_copy(
                        msg_hbm.at[pl.ds(base + (j + 1) * GCH, GCH)],
                        rows_a, sma)
                drain(rows_b, smb)
                pltpu.sync_copy(rows_b, shared.at[idx_v.at[j]], add=True)

        plsc.subcore_barrier()
        pltpu.sync_copy(shared.at[pl.ds(r0, rps)],
                        out_hbm.at[pl.ds(cid * n_pad + r0, rps)])

    return k(msg, idx.reshape(NW, n_ch, GCH), zeros)


def _matrelu_body(x_ref, w_ref, b_ref, o_ref):
    o_ref[...] = jnp.maximum(
        jnp.dot(x_ref[...], w_ref[...], preferred_element_type=jnp.float32)
        + b_ref[...], 0.0)


def _matrelu(x, w, b, out_dim, rb=8000):
    n, k = x.shape
    return pl.pallas_call(
        _matrelu_body,
        grid=(n // rb,),
        in_specs=[
            pl.BlockSpec((rb, k), lambda i: (i, 0)),
            pl.BlockSpec((k, out_dim), lambda i: (0, 0)),
            pl.BlockSpec((1, out_dim), lambda i: (0, 0)),
        ],
        out_specs=pl.BlockSpec((rb, out_dim), lambda i: (i, 0)),
        out_shape=jax.ShapeDtypeStruct((n, out_dim), jnp.float32),
    )(x, w, b.reshape(1, out_dim))


def _lin0_body(x_ref, w_ref, b_ref, o_ref, osc_ref):
    res = jnp.maximum(
        jnp.dot(x_ref[...], w_ref[...], preferred_element_type=jnp.float32)
        + b_ref[...], 0.0)
    o_ref[...] = res
    osc_ref[...] = jnp.concatenate([res, res], axis=1)


def _lin0(x, w, b):
    n = x.shape[0]
    return pl.pallas_call(
        _lin0_body,
        out_shape=(jax.ShapeDtypeStruct((n, D), jnp.float32),
                   jax.ShapeDtypeStruct((n, 2 * D), jnp.float32)),
    )(x, w, b.reshape(1, D))


def _msg_body(z_ref, hs_ref, w2r_ref, bm_ref, msg_ref):
    z = z_ref[...].astype(jnp.bfloat16)
    hs16 = hs_ref[:, :D].astype(jnp.bfloat16)
    u = jnp.concatenate([z[:, c:c + 1] * hs16 for c in range(32)], axis=1)
    acc = (jnp.dot(u, w2r_ref[...], preferred_element_type=jnp.float32)
           + jnp.dot(hs16, bm_ref[...], preferred_element_type=jnp.float32))
    msg_ref[...] = jnp.concatenate([acc, jnp.zeros_like(acc)], axis=1)


def _msg(z, hs, w2r, bmat, eb=3200):
    e = z.shape[0]
    return pl.pallas_call(
        _msg_body,
        grid=(e // eb,),
        in_specs=[
            pl.BlockSpec((eb, 32), lambda i: (i, 0)),
            pl.BlockSpec((eb, 2 * D), lambda i: (i, 0)),
            pl.BlockSpec((2048, D), lambda i: (0, 0)),
            pl.BlockSpec((D, D), lambda i: (0, 0)),
        ],
        out_specs=pl.BlockSpec((eb, 2 * D), lambda i: (i, 0)),
        out_shape=jax.ShapeDtypeStruct((e, 2 * D), jnp.float32),
    )(z, hs, w2r.astype(jnp.bfloat16), bmat.astype(jnp.bfloat16))


def _upd_body(agg2_ref, out_ref, wt_ref, wb_ref, cb_ref, mb_ref, new_ref,
              nsc_ref):
    n = out_ref.shape[0]
    n_pad = agg2_ref.shape[0] // 2
    agg = agg2_ref[:n, :D] + agg2_ref[n_pad:n_pad + n, :D]
    out = out_ref[...]
    m = jnp.maximum(agg + out + cb_ref[...], 0.0)
    res = (jnp.dot(m, wt_ref[...], preferred_element_type=jnp.float32)
           + jnp.dot(out, wb_ref[...], preferred_element_type=jnp.float32)
           + mb_ref[...])
    new_ref[...] = res
    nsc_ref[...] = jnp.concatenate([res, res], axis=1)


def _upd(agg2, out, wt, wb, cb, mb):
    n = out.shape[0]
    return pl.pallas_call(
        _upd_body,
        out_shape=(jax.ShapeDtypeStruct((n, D), jnp.float32),
                   jax.ShapeDtypeStruct((n, 2 * D), jnp.float32)),
    )(agg2, out, wt, wb, cb.reshape(1, D), mb.reshape(1, D))


def _ln_rows(x, g, b):
    mu = jnp.mean(x, axis=1, keepdims=True)
    xc = x - mu
    var = jnp.mean(xc * xc, axis=1, keepdims=True)
    return xc * lax.rsqrt(var + EPS) * g + b


def _proj_body(out_ref, init_ref, g_ref, b_ref, wq_ref, wk_ref, wv_ref,
               q_ref, k_ref, v_ref):
    h = _ln_rows(out_ref[...] + init_ref[...], g_ref[...], b_ref[...])
    q_ref[...] = jnp.maximum(
        jnp.dot(h, wq_ref[...], preferred_element_type=jnp.float32), 0.0)
    k_ref[...] = jnp.maximum(
        jnp.dot(h, wk_ref[...], preferred_element_type=jnp.float32), 0.0)
    v_ref[...] = jnp.dot(h, wv_ref[...], preferred_element_type=jnp.float32)


def _proj(out, init, g, b, wq, wk, wv):
    n = out.shape[0]
    sh = jax.ShapeDtypeStruct((n, D), jnp.float32)
    return pl.pallas_call(
        _proj_body,
        out_shape=(sh, sh, sh),
    )(out, init, g.reshape(1, D), b.reshape(1, D), wq, wk, wv)


def _attn_row_body(q_ref, k_ref, v_ref, mask_ref, g_ref, b_ref, c_ref):
    q = q_ref[...]
    s = lax.dot_general(q, k_ref[...], (((1,), (1,)), ((), ())),
                        preferred_element_type=jnp.float32)
    mask = mask_ref[...]
    a = mask * s - 1000.0 * (1.0 - mask)
    amax = jnp.max(a, axis=1, keepdims=True)
    ex = jnp.exp(a - amax)
    p = ex / jnp.sum(ex, axis=1, keepdims=True)
    c = jnp.dot(p, v_ref[...], preferred_element_type=jnp.float32)
    c_ref[...] = _ln_rows(c, g_ref[...], b_ref[...])


def _attn_row(q, k, v, mask, g, b, bq=400):
    nq = q.shape[0]
    nk = k.shape[0]
    return pl.pallas_call(
        _attn_row_body,
        grid=(nq // bq,),
        in_specs=[
            pl.BlockSpec((bq, D), lambda i: (i, 0)),
            pl.BlockSpec((nk, D), lambda i: (0, 0)),
            pl.BlockSpec((nk, D), lambda i: (0, 0)),
            pl.BlockSpec((bq, nk), lambda i: (i, 0)),
            pl.BlockSpec((1, D), lambda i: (0, 0)),
            pl.BlockSpec((1, D), lambda i: (0, 0)),
        ],
        out_specs=pl.BlockSpec((bq, D), lambda i: (i, 0)),
        out_shape=jax.ShapeDtypeStruct((nq, D), jnp.float32),
    )(q, k, v, mask, g.reshape(1, D), b.reshape(1, D))


def _attn_col_body(q_ref, k_ref, v_ref, mask_ref, g_ref, b_ref, c_ref):
    s = lax.dot_general(k_ref[...], q_ref[...], (((1,), (1,)), ((), ())),
                        preferred_element_type=jnp.float32)
    mask = mask_ref[...]
    a = mask * s - 1000.0 * (1.0 - mask)
    amax = jnp.max(a, axis=0, keepdims=True)
    ex = jnp.exp(a - amax)
    p = ex / jnp.sum(ex, axis=0, keepdims=True)
    c = lax.dot_general(p, v_ref[...], (((0,), (0,)), ((), ())),
                        preferred_element_type=jnp.float32)
    c_ref[...] = _ln_rows(c, g_ref[...], b_ref[...])


def _attn_col(q, k, v, mask, g, b, bq=512):
    nq = q.shape[0]
    nk = k.shape[0]
    return pl.pallas_call(
        _attn_col_body,
        grid=(nq // bq,),
        in_specs=[
            pl.BlockSpec((bq, D), lambda i: (i, 0)),
            pl.BlockSpec((nk, D), lambda i: (0, 0)),
            pl.BlockSpec((nk, D), lambda i: (0, 0)),
            pl.BlockSpec((nk, bq), lambda i: (0, i)),
            pl.BlockSpec((1, D), lambda i: (0, 0)),
            pl.BlockSpec((1, D), lambda i: (0, 0)),
        ],
        out_specs=pl.BlockSpec((bq, D), lambda i: (i, 0)),
        out_shape=jax.ShapeDtypeStruct((nq, D), jnp.float32),
    )(q, k, v, mask, g.reshape(1, D), b.reshape(1, D))


def _two_branches(x_l, ef_l, ei_l, x_r, ef_r, ei_r, p, w2r, bmat, wt, wb):
    z_l = _matrelu(ef_l, p['en_W1'], p['en_b1'], 32)
    z_r = _matrelu(ef_r, p['en_W1'], p['en_b1'], 32)
    out_l, sc_l = _lin0(x_l, p['lin0_W'], p['lin0_b'])
    out_r, sc_r = _lin0(x_r, p['lin0_W'], p['lin0_b'])
    n_l, n_r = x_l.shape[0], x_r.shape[0]
    np_l, np_r = -(-n_l // 128) * 128, -(-n_r // 128) * 128
    zeros_l = jnp.zeros((np_l, 2 * D), jnp.float32)
    zeros_r = jnp.zeros((np_r, 2 * D), jnp.float32)
    for _ in range(3):
        hs_l = _sc_gather(sc_l, ei_l[0])
        hs_r = _sc_gather(sc_r, ei_r[0])
        msg_l = _msg(z_l, hs_l, w2r, bmat)
        msg_r = _msg(z_r, hs_r, w2r, bmat)
        agg_l = _sc_scatter_add(msg_l, ei_l[1], zeros_l, np_l)
        agg_r = _sc_scatter_add(msg_r, ei_r[1], zeros_r, np_r)
        out_l, sc_l = _upd(agg_l, out_l, wt, wb, p['conv_b'], p['msg_b'])
        out_r, sc_r = _upd(agg_r, out_r, wt, wb, p['conv_b'], p['msg_b'])
    return out_l, out_r


def kernel(lig_n_feat, lig_e_feat, lig_edge_index, rec_n_feat, rec_e_feat,
           rec_edge_index, mask, params):
    p = params
    w2r = p['en_W2'].reshape(2048, D)
    bmat = p['en_b2'].reshape(D, D)
    wt, wb = p['msg_W'][:D], p['msg_W'][D:]

    out_l, out_r = _two_branches(lig_n_feat, lig_e_feat, lig_edge_index,
                                 rec_n_feat, rec_e_feat, rec_edge_index,
                                 p, w2r, bmat, wt, wb)

    q_l, k_l, v_l = _proj(out_l, lig_n_feat, p['ln_lig_g'], p['ln_lig_b'],
                          p['Wq_lig'], p['Wk_lig'], p['Wv_lig'])
    q_r, k_r, v_r = _proj(out_r, rec_n_feat, p['ln_rec_g'], p['ln_rec_b'],
                          p['Wq_rec'], p['Wk_rec'], p['Wv_rec'])

    n_rec = rec_n_feat.shape[0]
    nr_pad = -(-n_rec // 128) * 128
    padr = nr_pad - n_rec
    mask_p = jnp.pad(mask, ((0, 0), (0, padr)))
    c_l = _attn_row(q_l,
                    jnp.pad(k_r, ((0, padr), (0, 0))),
                    jnp.pad(v_r, ((0, padr), (0, 0))),
                    mask_p, p['ln_lig_g'], p['ln_lig_b'])
    c_r = _attn_col(jnp.pad(q_r, ((0, padr), (0, 0))),
                    k_l, v_l, mask_p,
                    p['ln_rec_g'], p['ln_rec_b'])[:n_rec]
    return jnp.concatenate([c_l, c_r], axis=0)


# speedup vs baseline: 1.3473x; 1.2892x over previous
import functools

import jax
import jax.numpy as jnp
from jax import lax
from jax.experimental import pallas as pl
from jax.experimental.pallas import tpu as pltpu
from jax.experimental.pallas import tpu_sc as plsc

D = 64
EPS = 1e-5
NC, NS = 2, 16
NW = NC * NS
GCH = 40


def _sc_gather(nodes, idx):
    e = idx.shape[0]
    per_w = e // NW
    n_ch = per_w // GCH
    mesh = plsc.VectorSubcoreMesh(core_axis_name="c", subcore_axis_name="s")

    @functools.partial(
        pl.kernel,
        mesh=mesh,
        out_type=jax.ShapeDtypeStruct((e, 2 * D), jnp.float32),
        scratch_types=[
            pltpu.VMEM((per_w,), jnp.int32),
            pltpu.VMEM((GCH, 2 * D), jnp.float32),
            pltpu.VMEM((GCH, 2 * D), jnp.float32),
            pltpu.SemaphoreType.DMA,
            pltpu.SemaphoreType.DMA,
        ],
    )
    def k(nodes_hbm, idx_hbm, out_hbm, idx_v, rows_a, rows_b, sga, sgb):
        wid = lax.axis_index("s") * NC + lax.axis_index("c")
        base = wid * per_w
        pltpu.sync_copy(idx_hbm.at[pl.ds(base, per_w)], idx_v)
        pltpu.async_copy(nodes_hbm.at[idx_v.at[pl.ds(0, GCH)]], rows_a, sga)

        def drain(rows, sem):
            pltpu.make_async_copy(nodes_hbm.at[pl.ds(0, GCH)], rows, sem).wait()

        @pl.loop(0, n_ch)
        def _(j):
            @pl.when(j % 2 == 0)
            def _():
                @pl.when(j + 1 < n_ch)
                def _():
                    pltpu.async_copy(
                        nodes_hbm.at[idx_v.at[pl.ds((j + 1) * GCH, GCH)]],
                        rows_b, sgb)
                drain(rows_a, sga)
                pltpu.sync_copy(rows_a, out_hbm.at[pl.ds(base + j * GCH, GCH)])

            @pl.when(j % 2 == 1)
            def _():
                @pl.when(j + 1 < n_ch)
                def _():
                    pltpu.async_copy(
                        nodes_hbm.at[idx_v.at[pl.ds((j + 1) * GCH, GCH)]],
                        rows_a, sga)
                drain(rows_b, sgb)
                pltpu.sync_copy(rows_b, out_hbm.at[pl.ds(base + j * GCH, GCH)])

    return k(nodes, idx)


def _sc_scatter_add(msg, idx, zeros, n_pad):
    e = idx.shape[0]
    per_w = e // NW
    n_ch = per_w // GCH
    rps = n_pad // NS
    mesh = plsc.VectorSubcoreMesh(core_axis_name="c", subcore_axis_name="s")

    @functools.partial(
        pl.kernel,
        mesh=mesh,
        out_type=jax.ShapeDtypeStruct((2 * n_pad, 2 * D), jnp.float32),
        scratch_types=[
            pltpu.VMEM((n_ch, GCH), jnp.int32),
            pltpu.VMEM((GCH, 2 * D), jnp.float32),
            pltpu.VMEM((GCH, 2 * D), jnp.float32),
            pltpu.VMEM_SHARED((n_pad, 2 * D), jnp.float32),
            pltpu.SemaphoreType.DMA,
            pltpu.SemaphoreType.DMA,
        ],
    )
    def k(msg_hbm, idx_hbm, zeros_hbm, out_hbm, idx_v, rows_a, rows_b, shared,
          sma, smb):
        cid = lax.axis_index("c")
        sid = lax.axis_index("s")
        wid = sid * NC + cid
        r0 = sid * rps

        pltpu.sync_copy(idx_hbm.at[wid], idx_v)
        pltpu.sync_copy(zeros_hbm.at[pl.ds(r0, rps)], shared.at[pl.ds(r0, rps)])
        plsc.subcore_barrier()

        base = wid * per_w
        pltpu.async_copy(msg_hbm.at[pl.ds(base, GCH)], rows_a, sma)

        def drain(rows, sem):
            pltpu.make_async_copy(msg_hbm.at[pl.ds(0, GCH)], rows, sem).wait()

        @pl.loop(0, n_ch)
        def _(j):
            @pl.when(j % 2 == 0)
            def _():
                @pl.when(j + 1 < n_ch)
                def _():
                    pltpu.async_copy(
                        msg_hbm.at[pl.ds(base + (j + 1) * GCH, GCH)],
                        rows_b, smb)
                drain(rows_a, sma)
                pltpu.sync_copy(rows_a, shared.at[idx_v.at[j]], add=True)

            @pl.when(j % 2 == 1)
            def _():
                @pl.when(j + 1 < n_ch)
                def _():
                    pltpu.async_copy(
                        msg_hbm.at[pl.ds(base + (j + 1) * GCH, GCH)],
                        rows_a, sma)
                drain(rows_b, smb)
                pltpu.sync_copy(rows_b, shared.at[idx_v.at[j]], add=True)

        plsc.subcore_barrier()
        pltpu.sync_copy(shared.at[pl.ds(r0, rps)],
                        out_hbm.at[pl.ds(cid * n_pad + r0, rps)])

    return k(msg, idx.reshape(NW, n_ch, GCH), zeros)


def _matrelu_body(x_ref, w_ref, b_ref, o_ref):
    o_ref[...] = jnp.maximum(
        jnp.dot(x_ref[...], w_ref[...], preferred_element_type=jnp.float32)
        + b_ref[...], 0.0)


def _matrelu(x, w, b, out_dim, rb=8000):
    n, k = x.shape
    return pl.pallas_call(
        _matrelu_body,
        grid=(n // rb,),
        in_specs=[
            pl.BlockSpec((rb, k), lambda i: (i, 0)),
            pl.BlockSpec((k, out_dim), lambda i: (0, 0)),
            pl.BlockSpec((1, out_dim), lambda i: (0, 0)),
        ],
        out_specs=pl.BlockSpec((rb, out_dim), lambda i: (i, 0)),
        out_shape=jax.ShapeDtypeStruct((n, out_dim), jnp.float32),
    )(x, w, b.reshape(1, out_dim))


def _lin0_body(x_ref, w_ref, b_ref, o_ref, osc_ref):
    res = jnp.maximum(
        jnp.dot(x_ref[...], w_ref[...], preferred_element_type=jnp.float32)
        + b_ref[...], 0.0)
    o_ref[...] = res
    osc_ref[...] = jnp.concatenate([res, res], axis=1)


def _lin0(x, w, b):
    n = x.shape[0]
    return pl.pallas_call(
        _lin0_body,
        out_shape=(jax.ShapeDtypeStruct((n, D), jnp.float32),
                   jax.ShapeDtypeStruct((n, 2 * D), jnp.float32)),
    )(x, w, b.reshape(1, D))


def _msg_body(z_ref, hs_ref, sel_ref, w2r_ref, bm_ref, msg_ref):
    z = z_ref[...].astype(jnp.bfloat16)
    hs2 = hs_ref[...].astype(jnp.bfloat16)
    zsel = jnp.dot(z, sel_ref[...],
                   preferred_element_type=jnp.float32).astype(jnp.bfloat16)
    u = zsel * jnp.concatenate([hs2] * 16, axis=1)
    acc = (jnp.dot(u, w2r_ref[...], preferred_element_type=jnp.float32)
           + jnp.dot(hs2[:, :D], bm_ref[...], preferred_element_type=jnp.float32))
    msg_ref[...] = jnp.concatenate([acc, jnp.zeros_like(acc)], axis=1)


def _sel_matrix():
    import numpy as np
    sel = np.zeros((32, 2048), np.float32)
    for k in range(16):
        sel[2 * k, 128 * k:128 * k + D] = 1.0
        sel[2 * k + 1, 128 * k + D:128 * k + 2 * D] = 1.0
    return jnp.asarray(sel, dtype=jnp.bfloat16)


def _msg(z, hs, w2r, bmat, eb=1600):
    e = z.shape[0]
    return pl.pallas_call(
        _msg_body,
        grid=(e // eb,),
        in_specs=[
            pl.BlockSpec((eb, 32), lambda i: (i, 0)),
            pl.BlockSpec((eb, 2 * D), lambda i: (i, 0)),
            pl.BlockSpec((32, 2048), lambda i: (0, 0)),
            pl.BlockSpec((2048, D), lambda i: (0, 0)),
            pl.BlockSpec((D, D), lambda i: (0, 0)),
        ],
        out_specs=pl.BlockSpec((eb, 2 * D), lambda i: (i, 0)),
        out_shape=jax.ShapeDtypeStruct((e, 2 * D), jnp.float32),
    )(z, hs, _sel_matrix(), w2r.astype(jnp.bfloat16), bmat.astype(jnp.bfloat16))


def _upd_body(agg2_ref, out_ref, wt_ref, wb_ref, cb_ref, mb_ref, new_ref,
              nsc_ref):
    n = out_ref.shape[0]
    n_pad = agg2_ref.shape[0] // 2
    agg = agg2_ref[:n, :D] + agg2_ref[n_pad:n_pad + n, :D]
    out = out_ref[...]
    m = jnp.maximum(agg + out + cb_ref[...], 0.0)
    res = (jnp.dot(m, wt_ref[...], preferred_element_type=jnp.float32)
           + jnp.dot(out, wb_ref[...], preferred_element_type=jnp.float32)
           + mb_ref[...])
    new_ref[...] = res
    nsc_ref[...] = jnp.concatenate([res, res], axis=1)


def _upd(agg2, out, wt, wb, cb, mb):
    n = out.shape[0]
    return pl.pallas_call(
        _upd_body,
        out_shape=(jax.ShapeDtypeStruct((n, D), jnp.float32),
                   jax.ShapeDtypeStruct((n, 2 * D), jnp.float32)),
    )(agg2, out, wt, wb, cb.reshape(1, D), mb.reshape(1, D))


def _ln_rows(x, g, b):
    mu = jnp.mean(x, axis=1, keepdims=True)
    xc = x - mu
    var = jnp.mean(xc * xc, axis=1, keepdims=True)
    return xc * lax.rsqrt(var + EPS) * g + b


def _proj_body(out_ref, init_ref, g_ref, b_ref, wq_ref, wk_ref, wv_ref,
               q_ref, k_ref, v_ref):
    h = _ln_rows(out_ref[...] + init_ref[...], g_ref[...], b_ref[...])
    q_ref[...] = jnp.maximum(
        jnp.dot(h, wq_ref[...], preferred_element_type=jnp.float32), 0.0)
    k_ref[...] = jnp.maximum(
        jnp.dot(h, wk_ref[...], preferred_element_type=jnp.float32), 0.0)
    v_ref[...] = jnp.dot(h, wv_ref[...], preferred_element_type=jnp.float32)


def _proj(out, init, g, b, wq, wk, wv):
    n = out.shape[0]
    sh = jax.ShapeDtypeStruct((n, D), jnp.float32)
    return pl.pallas_call(
        _proj_body,
        out_shape=(sh, sh, sh),
    )(out, init, g.reshape(1, D), b.reshape(1, D), wq, wk, wv)


def _attn_row_body(q_ref, k_ref, v_ref, mask_ref, g_ref, b_ref, c_ref):
    q = q_ref[...]
    s = lax.dot_general(q, k_ref[...], (((1,), (1,)), ((), ())),
                        preferred_element_type=jnp.float32)
    mask = mask_ref[...]
    a = mask * s - 1000.0 * (1.0 - mask)
    amax = jnp.max(a, axis=1, keepdims=True)
    ex = jnp.exp(a - amax)
    p = ex / jnp.sum(ex, axis=1, keepdims=True)
    c = jnp.dot(p, v_ref[...], preferred_element_type=jnp.float32)
    c_ref[...] = _ln_rows(c, g_ref[...], b_ref[...])


def _attn_row(q, k, v, mask, g, b, bq=400):
    nq = q.shape[0]
    nk = k.shape[0]
    return pl.pallas_call(
        _attn_row_body,
        grid=(nq // bq,),
        in_specs=[
            pl.BlockSpec((bq, D), lambda i: (i, 0)),
            pl.BlockSpec((nk, D), lambda i: (0, 0)),
            pl.BlockSpec((nk, D), lambda i: (0, 0)),
            pl.BlockSpec((bq, nk), lambda i: (i, 0)),
            pl.BlockSpec((1, D), lambda i: (0, 0)),
            pl.BlockSpec((1, D), lambda i: (0, 0)),
        ],
        out_specs=pl.BlockSpec((bq, D), lambda i: (i, 0)),
        out_shape=jax.ShapeDtypeStruct((nq, D), jnp.float32),
    )(q, k, v, mask, g.reshape(1, D), b.reshape(1, D))


def _attn_col_body(q_ref, k_ref, v_ref, mask_ref, g_ref, b_ref, c_ref):
    s = lax.dot_general(k_ref[...], q_ref[...], (((1,), (1,)), ((), ())),
                        preferred_element_type=jnp.float32)
    mask = mask_ref[...]
    a = mask * s - 1000.0 * (1.0 - mask)
    amax = jnp.max(a, axis=0, keepdims=True)
    ex = jnp.exp(a - amax)
    p = ex / jnp.sum(ex, axis=0, keepdims=True)
    c = lax.dot_general(p, v_ref[...], (((0,), (0,)), ((), ())),
                        preferred_element_type=jnp.float32)
    c_ref[...] = _ln_rows(c, g_ref[...], b_ref[...])


def _attn_col(q, k, v, mask, g, b, bq=512):
    nq = q.shape[0]
    nk = k.shape[0]
    return pl.pallas_call(
        _attn_col_body,
        grid=(nq // bq,),
        in_specs=[
            pl.BlockSpec((bq, D), lambda i: (i, 0)),
            pl.BlockSpec((nk, D), lambda i: (0, 0)),
            pl.BlockSpec((nk, D), lambda i: (0, 0)),
            pl.BlockSpec((nk, bq), lambda i: (0, i)),
            pl.BlockSpec((1, D), lambda i: (0, 0)),
            pl.BlockSpec((1, D), lambda i: (0, 0)),
        ],
        out_specs=pl.BlockSpec((bq, D), lambda i: (i, 0)),
        out_shape=jax.ShapeDtypeStruct((nq, D), jnp.float32),
    )(q, k, v, mask, g.reshape(1, D), b.reshape(1, D))


def _two_branches(x_l, ef_l, ei_l, x_r, ef_r, ei_r, p, w2r, bmat, wt, wb):
    z_l = _matrelu(ef_l, p['en_W1'], p['en_b1'], 32)
    z_r = _matrelu(ef_r, p['en_W1'], p['en_b1'], 32)
    out_l, sc_l = _lin0(x_l, p['lin0_W'], p['lin0_b'])
    out_r, sc_r = _lin0(x_r, p['lin0_W'], p['lin0_b'])
    n_l, n_r = x_l.shape[0], x_r.shape[0]
    np_l, np_r = -(-n_l // 128) * 128, -(-n_r // 128) * 128
    zeros_l = jnp.zeros((np_l, 2 * D), jnp.float32)
    zeros_r = jnp.zeros((np_r, 2 * D), jnp.float32)
    for _ in range(3):
        hs_l = _sc_gather(sc_l, ei_l[0])
        hs_r = _sc_gather(sc_r, ei_r[0])
        msg_l = _msg(z_l, hs_l, w2r, bmat)
        msg_r = _msg(z_r, hs_r, w2r, bmat)
        agg_l = _sc_scatter_add(msg_l, ei_l[1], zeros_l, np_l)
        agg_r = _sc_scatter_add(msg_r, ei_r[1], zeros_r, np_r)
        out_l, sc_l = _upd(agg_l, out_l, wt, wb, p['conv_b'], p['msg_b'])
        out_r, sc_r = _upd(agg_r, out_r, wt, wb, p['conv_b'], p['msg_b'])
    return out_l, out_r


def kernel(lig_n_feat, lig_e_feat, lig_edge_index, rec_n_feat, rec_e_feat,
           rec_edge_index, mask, params):
    p = params
    w2r = p['en_W2'].reshape(2048, D)
    bmat = p['en_b2'].reshape(D, D)
    wt, wb = p['msg_W'][:D], p['msg_W'][D:]

    out_l, out_r = _two_branches(lig_n_feat, lig_e_feat, lig_edge_index,
                                 rec_n_feat, rec_e_feat, rec_edge_index,
                                 p, w2r, bmat, wt, wb)

    q_l, k_l, v_l = _proj(out_l, lig_n_feat, p['ln_lig_g'], p['ln_lig_b'],
                          p['Wq_lig'], p['Wk_lig'], p['Wv_lig'])
    q_r, k_r, v_r = _proj(out_r, rec_n_feat, p['ln_rec_g'], p['ln_rec_b'],
                          p['Wq_rec'], p['Wk_rec'], p['Wv_rec'])

    n_rec = rec_n_feat.shape[0]
    nr_pad = -(-n_rec // 128) * 128
    padr = nr_pad - n_rec
    mask_p = jnp.pad(mask, ((0, 0), (0, padr)))
    c_l = _attn_row(q_l,
                    jnp.pad(k_r, ((0, padr), (0, 0))),
                    jnp.pad(v_r, ((0, padr), (0, 0))),
                    mask_p, p['ln_lig_g'], p['ln_lig_b'])
    c_r = _attn_col(jnp.pad(q_r, ((0, padr), (0, 0))),
                    k_l, v_l, mask_p,
                    p['ln_rec_g'], p['ln_rec_b'])[:n_rec]
    return jnp.concatenate([c_l, c_r], axis=0)


# speedup vs baseline: 1.3594x; 1.0089x over previous
import functools

import jax
import jax.numpy as jnp
from jax import lax
from jax.experimental import pallas as pl
from jax.experimental.pallas import tpu as pltpu
from jax.experimental.pallas import tpu_sc as plsc

D = 64
EPS = 1e-5
NC, NS = 2, 16
NW = NC * NS
GCH = 40


def _sc_gather(nodes, idx):
    e = idx.shape[0]
    per_w = e // NW
    gch = 128
    sizes = [gch] * (per_w // gch)
    if per_w % gch:
        sizes.append(per_w % gch)
    offs = [sum(sizes[:j]) for j in range(len(sizes))]
    n_ch = len(sizes)
    mesh = plsc.VectorSubcoreMesh(core_axis_name="c", subcore_axis_name="s")

    @functools.partial(
        pl.kernel,
        mesh=mesh,
        out_type=jax.ShapeDtypeStruct((e, 2 * D), jnp.float32),
        scratch_types=[
            pltpu.VMEM((per_w,), jnp.int32),
            pltpu.VMEM((gch, 2 * D), jnp.float32),
            pltpu.VMEM((gch, 2 * D), jnp.float32),
            pltpu.SemaphoreType.DMA,
            pltpu.SemaphoreType.DMA,
        ],
    )
    def k(nodes_hbm, idx_hbm, out_hbm, idx_v, rows_a, rows_b, sga, sgb):
        wid = lax.axis_index("s") * NC + lax.axis_index("c")
        base = wid * per_w
        pltpu.sync_copy(idx_hbm.at[pl.ds(base, per_w)], idx_v)
        bufs = [(rows_a, sga), (rows_b, sgb)]

        def start(j):
            rows, sem = bufs[j % 2]
            pltpu.async_copy(
                nodes_hbm.at[idx_v.at[pl.ds(offs[j], sizes[j])]],
                rows.at[pl.ds(0, sizes[j])], sem)

        start(0)
        for j in range(n_ch):
            if j + 1 < n_ch:
                start(j + 1)
            rows, sem = bufs[j % 2]
            dst = rows.at[pl.ds(0, sizes[j])]
            pltpu.make_async_copy(
                nodes_hbm.at[pl.ds(0, sizes[j])], dst, sem).wait()
            pltpu.sync_copy(dst, out_hbm.at[pl.ds(base + offs[j], sizes[j])])

    return k(nodes, idx)


def _sc_scatter_add(msg, idx, zeros, n_pad):
    e = idx.shape[0]
    per_w = e // NW
    n_ch = per_w // GCH
    rps = n_pad // NS
    mesh = plsc.VectorSubcoreMesh(core_axis_name="c", subcore_axis_name="s")

    @functools.partial(
        pl.kernel,
        mesh=mesh,
        out_type=jax.ShapeDtypeStruct((2 * n_pad, 2 * D), jnp.float32),
        scratch_types=[
            pltpu.VMEM((n_ch, GCH), jnp.int32),
            pltpu.VMEM((GCH, 2 * D), jnp.float32),
            pltpu.VMEM((GCH, 2 * D), jnp.float32),
            pltpu.VMEM_SHARED((n_pad, 2 * D), jnp.float32),
            pltpu.SemaphoreType.DMA,
            pltpu.SemaphoreType.DMA,
        ],
    )
    def k(msg_hbm, idx_hbm, zeros_hbm, out_hbm, idx_v, rows_a, rows_b, shared,
          sma, smb):
        cid = lax.axis_index("c")
        sid = lax.axis_index("s")
        wid = sid * NC + cid
        r0 = sid * rps

        pltpu.sync_copy(idx_hbm.at[wid], idx_v)
        pltpu.sync_copy(zeros_hbm.at[pl.ds(r0, rps)], shared.at[pl.ds(r0, rps)])
        plsc.subcore_barrier()

        base = wid * per_w
        pltpu.async_copy(msg_hbm.at[pl.ds(base, GCH)], rows_a, sma)

        def drain(rows, sem):
            pltpu.make_async_copy(msg_hbm.at[pl.ds(0, GCH)], rows, sem).wait()

        @pl.loop(0, n_ch)
        def _(j):
            @pl.when(j % 2 == 0)
            def _():
                @pl.when(j + 1 < n_ch)
                def _():
                    pltpu.async_copy(
                        msg_hbm.at[pl.ds(base + (j + 1) * GCH, GCH)],
                        rows_b, smb)
                drain(rows_a, sma)
                pltpu.sync_copy(rows_a, shared.at[idx_v.at[j]], add=True)

            @pl.when(j % 2 == 1)
            def _():
                @pl.when(j + 1 < n_ch)
                def _():
                    pltpu.async_copy(
                        msg_hbm.at[pl.ds(base + (j + 1) * GCH, GCH)],
                        rows_a, sma)
                drain(rows_b, smb)
                pltpu.sync_copy(rows_b, shared.at[idx_v.at[j]], add=True)

        plsc.subcore_barrier()
        pltpu.sync_copy(shared.at[pl.ds(r0, rps)],
                        out_hbm.at[pl.ds(cid * n_pad + r0, rps)])

    return k(msg, idx.reshape(NW, n_ch, GCH), zeros)


def _matrelu_body(x_ref, w_ref, b_ref, o_ref):
    o_ref[...] = jnp.maximum(
        jnp.dot(x_ref[...], w_ref[...], preferred_element_type=jnp.float32)
        + b_ref[...], 0.0)


def _matrelu(x, w, b, out_dim, rb=8000):
    n, k = x.shape
    return pl.pallas_call(
        _matrelu_body,
        grid=(n // rb,),
        in_specs=[
            pl.BlockSpec((rb, k), lambda i: (i, 0)),
            pl.BlockSpec((k, out_dim), lambda i: (0, 0)),
            pl.BlockSpec((1, out_dim), lambda i: (0, 0)),
        ],
        out_specs=pl.BlockSpec((rb, out_dim), lambda i: (i, 0)),
        out_shape=jax.ShapeDtypeStruct((n, out_dim), jnp.float32),
    )(x, w, b.reshape(1, out_dim))


def _lin0_body(x_ref, w_ref, b_ref, o_ref, osc_ref):
    res = jnp.maximum(
        jnp.dot(x_ref[...], w_ref[...], preferred_element_type=jnp.float32)
        + b_ref[...], 0.0)
    o_ref[...] = res
    osc_ref[...] = jnp.concatenate([res, res], axis=1)


def _lin0(x, w, b):
    n = x.shape[0]
    return pl.pallas_call(
        _lin0_body,
        out_shape=(jax.ShapeDtypeStruct((n, D), jnp.float32),
                   jax.ShapeDtypeStruct((n, 2 * D), jnp.float32)),
    )(x, w, b.reshape(1, D))


def _msg_body(z_ref, hs_ref, sel_ref, w2r_ref, bm_ref, msg_ref):
    z = z_ref[...].astype(jnp.bfloat16)
    hs2 = hs_ref[...].astype(jnp.bfloat16)
    zsel = jnp.dot(z, sel_ref[...],
                   preferred_element_type=jnp.float32).astype(jnp.bfloat16)
    u = zsel * jnp.concatenate([hs2] * 16, axis=1)
    acc = (jnp.dot(u, w2r_ref[...], preferred_element_type=jnp.float32)
           + jnp.dot(hs2[:, :D], bm_ref[...], preferred_element_type=jnp.float32))
    msg_ref[...] = jnp.concatenate([acc, jnp.zeros_like(acc)], axis=1)


def _sel_matrix():
    import numpy as np
    sel = np.zeros((32, 2048), np.float32)
    for k in range(16):
        sel[2 * k, 128 * k:128 * k + D] = 1.0
        sel[2 * k + 1, 128 * k + D:128 * k + 2 * D] = 1.0
    return jnp.asarray(sel, dtype=jnp.bfloat16)


def _msg(z, hs, w2r, bmat, eb=1600):
    e = z.shape[0]
    return pl.pallas_call(
        _msg_body,
        grid=(e // eb,),
        in_specs=[
            pl.BlockSpec((eb, 32), lambda i: (i, 0)),
            pl.BlockSpec((eb, 2 * D), lambda i: (i, 0)),
            pl.BlockSpec((32, 2048), lambda i: (0, 0)),
            pl.BlockSpec((2048, D), lambda i: (0, 0)),
            pl.BlockSpec((D, D), lambda i: (0, 0)),
        ],
        out_specs=pl.BlockSpec((eb, 2 * D), lambda i: (i, 0)),
        out_shape=jax.ShapeDtypeStruct((e, 2 * D), jnp.float32),
    )(z, hs, _sel_matrix(), w2r.astype(jnp.bfloat16), bmat.astype(jnp.bfloat16))


def _upd_body(agg2_ref, out_ref, wt_ref, wb_ref, cb_ref, mb_ref, new_ref,
              nsc_ref):
    n = out_ref.shape[0]
    n_pad = agg2_ref.shape[0] // 2
    agg = agg2_ref[:n, :D] + agg2_ref[n_pad:n_pad + n, :D]
    out = out_ref[...]
    m = jnp.maximum(agg + out + cb_ref[...], 0.0)
    res = (jnp.dot(m, wt_ref[...], preferred_element_type=jnp.float32)
           + jnp.dot(out, wb_ref[...], preferred_element_type=jnp.float32)
           + mb_ref[...])
    new_ref[...] = res
    nsc_ref[...] = jnp.concatenate([res, res], axis=1)


def _upd(agg2, out, wt, wb, cb, mb):
    n = out.shape[0]
    return pl.pallas_call(
        _upd_body,
        out_shape=(jax.ShapeDtypeStruct((n, D), jnp.float32),
                   jax.ShapeDtypeStruct((n, 2 * D), jnp.float32)),
    )(agg2, out, wt, wb, cb.reshape(1, D), mb.reshape(1, D))


def _ln_rows(x, g, b):
    mu = jnp.mean(x, axis=1, keepdims=True)
    xc = x - mu
    var = jnp.mean(xc * xc, axis=1, keepdims=True)
    return xc * lax.rsqrt(var + EPS) * g + b


def _proj_body(out_ref, init_ref, g_ref, b_ref, wq_ref, wk_ref, wv_ref,
               q_ref, k_ref, v_ref):
    h = _ln_rows(out_ref[...] + init_ref[...], g_ref[...], b_ref[...])
    q_ref[...] = jnp.maximum(
        jnp.dot(h, wq_ref[...], preferred_element_type=jnp.float32), 0.0)
    k_ref[...] = jnp.maximum(
        jnp.dot(h, wk_ref[...], preferred_element_type=jnp.float32), 0.0)
    v_ref[...] = jnp.dot(h, wv_ref[...], preferred_element_type=jnp.float32)


def _proj(out, init, g, b, wq, wk, wv):
    n = out.shape[0]
    sh = jax.ShapeDtypeStruct((n, D), jnp.float32)
    return pl.pallas_call(
        _proj_body,
        out_shape=(sh, sh, sh),
    )(out, init, g.reshape(1, D), b.reshape(1, D), wq, wk, wv)


def _attn_row_body(q_ref, k_ref, v_ref, mask_ref, g_ref, b_ref, c_ref):
    q = q_ref[...]
    s = lax.dot_general(q, k_ref[...], (((1,), (1,)), ((), ())),
                        preferred_element_type=jnp.float32)
    mask = mask_ref[...]
    a = mask * s - 1000.0 * (1.0 - mask)
    amax = jnp.max(a, axis=1, keepdims=True)
    ex = jnp.exp(a - amax)
    p = ex / jnp.sum(ex, axis=1, keepdims=True)
    c = jnp.dot(p, v_ref[...], preferred_element_type=jnp.float32)
    c_ref[...] = _ln_rows(c, g_ref[...], b_ref[...])


def _attn_row(q, k, v, mask, g, b, bq=400):
    nq = q.shape[0]
    nk = k.shape[0]
    return pl.pallas_call(
        _attn_row_body,
        grid=(nq // bq,),
        in_specs=[
            pl.BlockSpec((bq, D), lambda i: (i, 0)),
            pl.BlockSpec((nk, D), lambda i: (0, 0)),
            pl.BlockSpec((nk, D), lambda i: (0, 0)),
            pl.BlockSpec((bq, nk), lambda i: (i, 0)),
            pl.BlockSpec((1, D), lambda i: (0, 0)),
            pl.BlockSpec((1, D), lambda i: (0, 0)),
        ],
        out_specs=pl.BlockSpec((bq, D), lambda i: (i, 0)),
        out_shape=jax.ShapeDtypeStruct((nq, D), jnp.float32),
    )(q, k, v, mask, g.reshape(1, D), b.reshape(1, D))


def _attn_col_body(q_ref, k_ref, v_ref, mask_ref, g_ref, b_ref, c_ref):
    s = lax.dot_general(k_ref[...], q_ref[...], (((1,), (1,)), ((), ())),
                        preferred_element_type=jnp.float32)
    mask = mask_ref[...]
    a = mask * s - 1000.0 * (1.0 - mask)
    amax = jnp.max(a, axis=0, keepdims=True)
    ex = jnp.exp(a - amax)
    p = ex / jnp.sum(ex, axis=0, keepdims=True)
    c = lax.dot_general(p, v_ref[...], (((0,), (0,)), ((), ())),
                        preferred_element_type=jnp.float32)
    c_ref[...] = _ln_rows(c, g_ref[...], b_ref[...])


def _attn_col(q, k, v, mask, g, b, bq=512):
    nq = q.shape[0]
    nk = k.shape[0]
    return pl.pallas_call(
        _attn_col_body,
        grid=(nq // bq,),
        in_specs=[
            pl.BlockSpec((bq, D), lambda i: (i, 0)),
            pl.BlockSpec((nk, D), lambda i: (0, 0)),
            pl.BlockSpec((nk, D), lambda i: (0, 0)),
            pl.BlockSpec((nk, bq), lambda i: (0, i)),
            pl.BlockSpec((1, D), lambda i: (0, 0)),
            pl.BlockSpec((1, D), lambda i: (0, 0)),
        ],
        out_specs=pl.BlockSpec((bq, D), lambda i: (i, 0)),
        out_shape=jax.ShapeDtypeStruct((nq, D), jnp.float32),
    )(q, k, v, mask, g.reshape(1, D), b.reshape(1, D))


def _two_branches(x_l, ef_l, ei_l, x_r, ef_r, ei_r, p, w2r, bmat, wt, wb):
    z_l = _matrelu(ef_l, p['en_W1'], p['en_b1'], 32)
    z_r = _matrelu(ef_r, p['en_W1'], p['en_b1'], 32)
    out_l, sc_l = _lin0(x_l, p['lin0_W'], p['lin0_b'])
    out_r, sc_r = _lin0(x_r, p['lin0_W'], p['lin0_b'])
    n_l, n_r = x_l.shape[0], x_r.shape[0]
    np_l, np_r = -(-n_l // 128) * 128, -(-n_r // 128) * 128
    zeros_l = jnp.zeros((np_l, 2 * D), jnp.float32)
    zeros_r = jnp.zeros((np_r, 2 * D), jnp.float32)
    for _ in range(3):
        hs_l = _sc_gather(sc_l, ei_l[0])
        hs_r = _sc_gather(sc_r, ei_r[0])
        msg_l = _msg(z_l, hs_l, w2r, bmat)
        msg_r = _msg(z_r, hs_r, w2r, bmat)
        agg_l = _sc_scatter_add(msg_l, ei_l[1], zeros_l, np_l)
        agg_r = _sc_scatter_add(msg_r, ei_r[1], zeros_r, np_r)
        out_l, sc_l = _upd(agg_l, out_l, wt, wb, p['conv_b'], p['msg_b'])
        out_r, sc_r = _upd(agg_r, out_r, wt, wb, p['conv_b'], p['msg_b'])
    return out_l, out_r


def kernel(lig_n_feat, lig_e_feat, lig_edge_index, rec_n_feat, rec_e_feat,
           rec_edge_index, mask, params):
    p = params
    w2r = p['en_W2'].reshape(2048, D)
    bmat = p['en_b2'].reshape(D, D)
    wt, wb = p['msg_W'][:D], p['msg_W'][D:]

    out_l, out_r = _two_branches(lig_n_feat, lig_e_feat, lig_edge_index,
                                 rec_n_feat, rec_e_feat, rec_edge_index,
                                 p, w2r, bmat, wt, wb)

    q_l, k_l, v_l = _proj(out_l, lig_n_feat, p['ln_lig_g'], p['ln_lig_b'],
                          p['Wq_lig'], p['Wk_lig'], p['Wv_lig'])
    q_r, k_r, v_r = _proj(out_r, rec_n_feat, p['ln_rec_g'], p['ln_rec_b'],
                          p['Wq_rec'], p['Wk_rec'], p['Wv_rec'])

    n_rec = rec_n_feat.shape[0]
    nr_pad = -(-n_rec // 128) * 128
    padr = nr_pad - n_rec
    mask_p = jnp.pad(mask, ((0, 0), (0, padr)))
    c_l = _attn_row(q_l,
                    jnp.pad(k_r, ((0, padr), (0, 0))),
                    jnp.pad(v_r, ((0, padr), (0, 0))),
                    mask_p, p['ln_lig_g'], p['ln_lig_b'])
    c_r = _attn_col(jnp.pad(q_r, ((0, padr), (0, 0))),
                    k_l, v_l, mask_p,
                    p['ln_rec_g'], p['ln_rec_b'])[:n_rec]
    return jnp.concatenate([c_l, c_r], axis=0)
